# Initial kernel scaffold; baseline (speedup 1.0000x reference)
import jax
import jax.numpy as jnp
from jax.experimental import pallas as pl


def kernel(coor, feat, W, bias):
    raise NotImplementedError("write your pallas kernel here")


# speedup vs baseline: 8.1366x; 8.1366x over previous
import dataclasses
import functools

import numpy as np
import jax

jax.config.update("jax_enable_x64", True)

import jax.numpy as jnp
from jax import lax
from jax.experimental import pallas as pl
from jax.experimental.pallas import tpu as pltpu
from jax.experimental.pallas import tpu_sc as plsc

N = 100000
D = 128
B = 4
G = 128
SIZE = 64
NVOX = 40000
TAB = 40960
DUMP = 40950
ZSLOT = 40960
TROWS = 41088
NB = 800
NREAL = 782
NK = 25
LASTB = 781
FTAIL = 99872
SRNG = 8192
SROWS = 8320
RES64 = 2 * 0.4

def _mesh():
    return plsc.VectorSubcoreMesh(core_axis_name="c", subcore_axis_name="s")

_no_layout_cp = pltpu.CompilerParams()
if "needs_layout_passes" in pltpu.CompilerParams.__dataclass_fields__:
    _no_layout_cp = dataclasses.replace(_no_layout_cp, needs_layout_passes=False)


def _slot_ids(bv, xv, yv):
    return bv * 10000 + (xv + 50) * 100 + (yv + 50)


def _tc_count_body(bp, xp, yp, cnt_ref):
    i = pl.program_id(0)

    @pl.when(i == 0)
    def _():
        cnt_ref[...] = jnp.zeros((320, 128), jnp.float32)

    hi_bins = lax.broadcasted_iota(jnp.int32, (320, 1), 0)
    lo_bins = lax.broadcasted_iota(jnp.int32, (128, 1), 0)
    acc = cnt_ref[...]
    for j in range(8):
        did = _slot_ids(bp[j:j + 1, :], xp[j:j + 1, :], yp[j:j + 1, :])
        did = jnp.minimum(did, DUMP)
        hi = did // 128
        lo = did - hi * 128
        H = (hi == hi_bins).astype(jnp.float32)
        L = (lo == lo_bins).astype(jnp.float32)
        acc = acc + lax.dot_general(H, L, (((1,), (1,)), ((), ())),
                                    preferred_element_type=jnp.float32)
    cnt_ref[...] = acc


def _tc_count(bp, xp, yp):
    spec = pl.BlockSpec((8, 128), lambda i: (i, 0))
    return pl.pallas_call(
        _tc_count_body,
        grid=(NB // 8,),
        in_specs=[spec, spec, spec],
        out_specs=pl.BlockSpec((320, 128), lambda i: (0, 0)),
        out_shape=jax.ShapeDtypeStruct((320, 128), jnp.float32),
    )(bp, xp, yp)


def _sc_scatter(bp, xp, yp, feat):
    @functools.partial(
        pl.kernel,
        mesh=_mesh(),
        out_type=jax.ShapeDtypeStruct((2, TAB, 128), jnp.float32),
        scratch_types=[
            pltpu.VMEM((NK, 128), jnp.int32),
            pltpu.VMEM((2, 128), jnp.int32),
            pltpu.VMEM((1, 128), jnp.int32),
            pltpu.VMEM((1, 128), jnp.int32),
            pltpu.VMEM((1, 128), jnp.int32),
            pltpu.VMEM((128, 128), jnp.float32),
            pltpu.VMEM((128, 128), jnp.float32),
            pltpu.VMEM((128, 128), jnp.float32),
            pltpu.VMEM_SHARED((SROWS, 128), jnp.float32),
            pltpu.SemaphoreType.DMA,
            pltpu.SemaphoreType.DMA,
            pltpu.SemaphoreType.DMA,
            pltpu.SemaphoreType.DMA,
        ],
    )
    def k(bp_h, xp_h, yp_h, feat_h, fpart_h,
          didb, lidb, bb, xb, yb, valb0, valb1, zb, ftab,
          lsem0, lsem1, ssem0, ssem1):
        cid = lax.axis_index("c").astype(jnp.int32)
        sid = lax.axis_index("s").astype(jnp.int32)
        wid = sid * 2 + cid
        valb = (valb0, valb1)
        lsem = (lsem0, lsem1)
        ssem = (ssem0, ssem1)

        @pl.loop(0, 128)
        def _init(r):
            r = r.astype(jnp.int32)
            for j in range(8):
                zb[r, pl.ds(16 * j, 16)] = jnp.zeros((16,), jnp.float32)

        @pl.loop(0, NK)
        def _ids(kk):
            kk = kk.astype(jnp.int32)
            bid = kk * 32 + wid
            pltpu.sync_copy(bp_h.at[pl.ds(bid, 1), :], bb)
            pltpu.sync_copy(xp_h.at[pl.ds(bid, 1), :], xb)
            pltpu.sync_copy(yp_h.at[pl.ds(bid, 1), :], yb)
            for j in range(8):
                did = _slot_ids(bb[0, pl.ds(16 * j, 16)],
                                xb[0, pl.ds(16 * j, 16)],
                                yb[0, pl.ds(16 * j, 16)])
                didb[kk, pl.ds(16 * j, 16)] = jnp.minimum(did, DUMP)

        def feat_src(kk):
            bid = kk * 32 + wid
            fb = jnp.minimum(bid * 128, FTAIL)
            return feat_h.at[pl.ds(fb, 128), :]

        def load_start(kk, a):
            pltpu.async_copy(feat_src(kk), valb[a], lsem[a])

        def load_wait(kk, a):
            pltpu.make_async_copy(feat_src(kk), valb[a], lsem[a]).wait()

        def scat_start(kk, a, p):
            for j in range(8):
                lid = didb[kk, pl.ds(16 * j, 16)] - p * SRNG
                ok = jnp.logical_and(lid >= 0, lid < SRNG)
                lidb[a, pl.ds(16 * j, 16)] = jnp.where(ok, lid, SRNG)
            pltpu.async_copy(valb[a], ftab.at[lidb.at[a]], ssem[a], add=True)

        def scat_wait(a):
            pltpu.make_async_copy(valb[a], ftab.at[lidb.at[a]], ssem[a]).wait()

        for p in range(5):
            base = sid * 520

            @pl.loop(0, 4)
            def _zero(z):
                z = z.astype(jnp.int32)
                pltpu.sync_copy(zb, ftab.at[pl.ds(base + z * 128, 128), :])

            pltpu.sync_copy(zb.at[pl.ds(0, 8), :],
                            ftab.at[pl.ds(base + 512, 8), :])
            plsc.subcore_barrier()

            load_start(0, 0)

            @pl.loop(0, NK // 2)
            def _scat(t):
                t = t.astype(jnp.int32)
                for b01 in range(2):
                    kk = t * 2 + b01
                    load_wait(kk, b01)
                    scat_start(kk, b01, p)

                    @pl.when(kk >= 1)
                    def _():
                        scat_wait(1 - b01)

                    @pl.when(kk + 1 < NK)
                    def _():
                        load_start(kk + 1, 1 - b01)

            load_wait(NK - 1, 0)
            scat_start(NK - 1, 0, p)
            scat_wait(1)
            scat_wait(0)

            plsc.subcore_barrier()
            pltpu.sync_copy(
                ftab.at[pl.ds(sid * 512, 512), :],
                fpart_h.at[cid, pl.ds(p * SRNG + sid * 512, 512), :])
            plsc.subcore_barrier()

    return k(bp, xp, yp, feat)


def _tc_tables_body(fp0, fp1, cnt_row, w_ref, b_ref, tmean_ref, aux_ref):
    i = pl.program_id(0)
    valid_block = i < 320
    slots = i * 128 + lax.broadcasted_iota(jnp.int32, (128, 1), 0)

    rI = lax.broadcasted_iota(jnp.int32, (128, 128), 0)
    cI = lax.broadcasted_iota(jnp.int32, (128, 128), 1)
    eye = jnp.where(rI == cI, 1.0, 0.0).astype(jnp.float32)
    cnt = lax.dot_general(eye, cnt_row[0], (((1,), (1,)), ((), ())),
                          preferred_element_type=jnp.float32)

    fsum = fp0[...] + fp1[...]
    inv = 1.0 / jnp.maximum(cnt, 1.0)
    tmean = fsum * inv
    tmean = jnp.where(valid_block, tmean, 0.0)
    tmean_ref[...] = tmean

    reg = jnp.dot(tmean, w_ref[...], preferred_element_type=jnp.float32)
    reg = reg + b_ref[...]

    invox = jnp.logical_and(slots < NVOX, valid_block)
    bq = slots // 10000
    rr = slots - bq * 10000
    xq = rr // 100 - 50
    yq = rr - (rr // 100) * 100 - 50
    vmask = jnp.where(invox, 1.0, 0.0).astype(jnp.float32)
    cb = bq.astype(jnp.float32) * vmask
    cx = xq.astype(jnp.float32) * jnp.float32(0.4) * vmask
    cy = yq.astype(jnp.float32) * jnp.float32(0.4) * vmask
    aux_ref[...] = jnp.concatenate(
        [reg, cb, cx, cy, jnp.zeros((128, 123), jnp.float32)], axis=1)


def _tc_tables(fpart, cnt, W, bias):
    clamp = lambda i: (jnp.minimum(i, 319), 0)
    return pl.pallas_call(
        _tc_tables_body,
        grid=(321,),
        in_specs=[
            pl.BlockSpec((128, 128), clamp),
            pl.BlockSpec((128, 128), clamp),
            pl.BlockSpec((1, 1, 128), lambda i: (jnp.minimum(i, 319), 0, 0)),
            pl.BlockSpec((128, 2), lambda i: (0, 0)),
            pl.BlockSpec((1, 2), lambda i: (0, 0)),
        ],
        out_specs=[
            pl.BlockSpec((128, 128), lambda i: (i, 0)),
            pl.BlockSpec((128, 128), lambda i: (i, 0)),
        ],
        out_shape=[
            jax.ShapeDtypeStruct((TROWS, 128), jnp.float32),
            jax.ShapeDtypeStruct((TROWS, 128), jnp.float32),
        ],
    )(fpart[0], fpart[1], cnt.reshape(320, 1, 128), W, bias.reshape(1, 2))


def _tc_obs_body(cg_ref, aall_ref, amid_ref, obs_ref):
    occ = (cg_ref[...] > 0.0).astype(jnp.float32)
    aall = aall_ref[...]
    amid = amid_ref[...]
    for b in range(B):
        m = occ[b]
        t1 = lax.dot_general(amid, m, (((0,), (0,)), ((), ())),
                             preferred_element_type=jnp.float32)
        t1 = jnp.dot(t1, aall, preferred_element_type=jnp.float32)
        t2 = lax.dot_general(aall, m, (((0,), (0,)), ((), ())),
                             preferred_element_type=jnp.float32)
        t2 = jnp.dot(t2, amid, preferred_element_type=jnp.float32)
        obs_ref[b] = ((t1 + t2) > 0.0).astype(jnp.int8)


def _tc_obs(cgrid, A_all, A_mid):
    return pl.pallas_call(
        _tc_obs_body,
        out_shape=jax.ShapeDtypeStruct((B, G, G), jnp.int8),
    )(cgrid, A_all, A_mid)


def _axis_tables(rt_zero):
    offs = jnp.asarray(np.linspace(-1.6, 1.6, 5), dtype=jnp.float32)
    xs = jnp.arange(-50, 50, dtype=jnp.float32) + rt_zero
    cell = jnp.floor((xs[:, None] + offs[None, :]) / RES64) + SIZE
    cell = cell.astype(jnp.int32)
    valid = (cell >= 0) & (cell < G)
    onehot = (cell[:, :, None] == jnp.arange(G, dtype=jnp.int32)) & valid[:, :, None]
    A_all = jnp.any(onehot, axis=1).astype(jnp.float32)
    A_mid = jnp.any(onehot[:, 1:4], axis=1).astype(jnp.float32)
    return A_all, A_mid


def _sc_compact(cnt):
    @functools.partial(
        pl.kernel,
        mesh=_mesh(),
        compiler_params=_no_layout_cp,
        out_type=jax.ShapeDtypeStruct((NREAL * 128,), jnp.int32),
        scratch_types=[
            pltpu.VMEM((NREAL * 128,), jnp.int32),
            pltpu.VMEM((160, 128), jnp.float32),
        ],
    )
    def k(cnt_h, sid_h, sidtab, cb):
        cid = lax.axis_index("c").astype(jnp.int32)
        sid = lax.axis_index("s").astype(jnp.int32)
        wid = sid * 2 + cid

        @pl.when(wid == 0)
        def _():
            @pl.loop(0, NREAL * 8)
            def _fill(j):
                j = j.astype(jnp.int32)
                sidtab[pl.ds(j * 16, 16)] = jnp.full((16,), ZSLOT, jnp.int32)

            def stage(r0, nrows, carry):
                pltpu.sync_copy(cnt_h.at[pl.ds(r0, nrows), :],
                                cb.at[pl.ds(0, nrows), :])

                def row(r, car):
                    c2 = car
                    for j in range(8):
                        ov = cb[r, pl.ds(j * 16, 16)]
                        ids = ((r0 + r) * 128 + j * 16
                               + jnp.arange(16, dtype=jnp.int32))
                        m = jnp.logical_and(ov > 0.0, ids < NVOX)
                        mi = jnp.where(m, 1, 0).astype(jnp.int32)
                        cs = plsc.cumsum(mi)
                        pos = c2 + cs - mi
                        plsc.store_scatter(sidtab, [pos], ids, mask=m)
                        c2 = c2 + jnp.sum(mi, dtype=jnp.int32)
                    return c2

                return lax.fori_loop(jnp.int32(0), jnp.int32(nrows), row, carry)

            carry = stage(0, 160, jnp.int32(0))
            carry = stage(160, 160, carry)
            pltpu.sync_copy(sidtab, sid_h)

    return k(cnt)


def _sc_gather(sidx, tmean, aux):
    @functools.partial(
        pl.kernel,
        mesh=_mesh(),
        out_type=(
            jax.ShapeDtypeStruct((N, 128), jnp.float32),
            jax.ShapeDtypeStruct((N, 16), jnp.float32),
        ),
        scratch_types=[
            pltpu.VMEM((128,), jnp.int32),
            pltpu.VMEM((128, 128), jnp.float32),
            pltpu.VMEM((128, 128), jnp.float32),
            pltpu.VMEM((128, 16), jnp.float32),
        ],
    )
    def k(sid_h, tmean_h, aux_h, fm_h, ax_h, idxb, rowb, auxb, packb):
        cid = lax.axis_index("c").astype(jnp.int32)
        sid = lax.axis_index("s").astype(jnp.int32)
        wid = sid * 2 + cid

        @pl.loop(0, NK)
        def _g(kk):
            kk = kk.astype(jnp.int32)
            bid = kk * 32 + wid

            @pl.when(bid < NREAL)
            def _():
                pltpu.sync_copy(sid_h.at[pl.ds(bid * 128, 128)], idxb)
                pltpu.sync_copy(tmean_h.at[idxb], rowb)
                pltpu.sync_copy(aux_h.at[idxb], auxb)

                @pl.loop(0, 128)
                def _pack(r):
                    r = r.astype(jnp.int32)
                    packb[r] = auxb[r, pl.ds(0, 16)]

                @pl.when(bid < LASTB)
                def _full():
                    pltpu.sync_copy(rowb, fm_h.at[pl.ds(bid * 128, 128), :])
                    pltpu.sync_copy(packb, ax_h.at[pl.ds(bid * 128, 128), :])

                @pl.when(bid == LASTB)
                def _tail():
                    pltpu.sync_copy(rowb.at[pl.ds(0, 32), :],
                                    fm_h.at[pl.ds(LASTB * 128, 32), :])
                    pltpu.sync_copy(packb.at[pl.ds(0, 32), :],
                                    ax_h.at[pl.ds(LASTB * 128, 32), :])

    return k(sidx, tmean, aux)


def kernel(coor, feat, W, bias):
    with jax.enable_x64(False):
        return _kernel_32(coor, feat, W, bias)


def _kernel_32(coor, feat, W, bias):
    b32 = coor[:, 0].astype(jnp.int32)
    x32 = coor[:, 1].astype(jnp.int32)
    y32 = coor[:, 2].astype(jnp.int32)

    def lanes(v, padval):
        head, tail = v[:LASTB * 128], v[LASTB * 128:]
        padmid = jnp.full((96,), padval, jnp.int32)
        padend = jnp.full((NB * 128 - NREAL * 128,), padval, jnp.int32)
        return jnp.concatenate([head, padmid, tail, padend]).reshape(NB, 128)

    bp = lanes(b32, 127)
    xp = lanes(x32, 0)
    yp = lanes(y32, 0)

    cnt = _tc_count(bp, xp, yp)
    fpart = _sc_scatter(bp, xp, yp, feat)
    tmean, aux = _tc_tables(fpart, cnt, W, bias)

    cgrid = cnt.reshape(-1)[:NVOX].reshape(B, 100, 100)
    A_all, A_mid = _axis_tables(feat[0, 0] * 0.0)
    obs8 = _tc_obs(cgrid, A_all, A_mid)
    obs_mask = obs8.astype(jnp.bool_)

    sidx = _sc_compact(cnt)
    fm, ax = _sc_gather(sidx, tmean, aux)

    reg = ax[:, 0:2]
    centers = ax[:, 2:5]
    return reg, obs_mask, centers, fm
